# Initial kernel scaffold; baseline (speedup 1.0000x reference)
import jax
import jax.numpy as jnp
from jax.experimental import pallas as pl


def kernel(x, edge_index, batch_idx, W0, b0, gamma0, beta0, W1, b1, gamma1, beta1, W2, b2, gamma2, beta2):
    raise NotImplementedError("write your pallas kernel here")


# speedup vs baseline: 11.2808x; 11.2808x over previous
import jax
import jax.numpy as jnp
from jax import lax
from jax.experimental import pallas as pl
from jax.experimental.pallas import tpu as pltpu
from jax.experimental.pallas import tpu_sc as plsc

N = 10000
D = 128
G = 64
NC = 2
NS = 16
NW = NC * NS
NP = 10112
RPS = NP // NS
CHUNK = 128


_MESH = plsc.VectorSubcoreMesh(core_axis_name="c", subcore_axis_name="s")


def _sc_deg_body(dst_hbm, zeros16_hbm, ones16_hbm, out_hbm, dst_v, ones_v, acc_sh):
    c = lax.axis_index("c")
    s = lax.axis_index("s")
    w = c * NS + s
    cpt = dst_hbm.shape[1]
    pltpu.sync_copy(dst_hbm.at[w], dst_v)
    pltpu.sync_copy(ones16_hbm, ones_v)
    pltpu.sync_copy(zeros16_hbm.at[pl.ds(s * RPS, RPS)], acc_sh.at[pl.ds(s * RPS, RPS)])
    plsc.subcore_barrier()

    def body(j, carry):
        pltpu.sync_copy(ones_v, acc_sh.at[dst_v.at[j]], add=True)
        return carry

    lax.fori_loop(0, cpt, body, 0)
    plsc.subcore_barrier()
    pltpu.sync_copy(acc_sh.at[pl.ds(s * RPS, RPS)], out_hbm.at[c, pl.ds(s * RPS, RPS)])


def _sc_deg(dst_tiles, zeros16, ones16):
    cpt = dst_tiles.shape[1]
    return pl.kernel(
        _sc_deg_body,
        out_type=jax.ShapeDtypeStruct((NC, NP, 16), jnp.float32),
        mesh=_MESH,
        scratch_types=[
            pltpu.VMEM((cpt, CHUNK), jnp.int32),
            pltpu.VMEM((CHUNK, 16), jnp.float32),
            pltpu.VMEM_SHARED((NP, 16), jnp.float32),
        ],
    )(dst_tiles, zeros16, ones16)


def _sc_agg_body(g_hbm, src_hbm, dst_hbm, zeros_hbm, out_hbm,
                 src_v, dst_v, rows_v, acc_sh, sem):
    c = lax.axis_index("c")
    s = lax.axis_index("s")
    w = c * NS + s
    cpt = src_hbm.shape[1]
    pltpu.sync_copy(src_hbm.at[w], src_v)
    pltpu.sync_copy(dst_hbm.at[w], dst_v)
    pltpu.sync_copy(zeros_hbm.at[pl.ds(s * RPS, RPS)], acc_sh.at[pl.ds(s * RPS, RPS)])
    plsc.subcore_barrier()

    def body(j, carry):
        pltpu.async_copy(g_hbm.at[src_v.at[j]], rows_v, sem).wait()
        pltpu.sync_copy(rows_v, acc_sh.at[dst_v.at[j]], add=True)
        return carry

    lax.fori_loop(0, cpt, body, 0)
    plsc.subcore_barrier()
    pltpu.sync_copy(acc_sh.at[pl.ds(s * RPS, RPS)], out_hbm.at[c, pl.ds(s * RPS, RPS)])


def _sc_agg(g, src_tiles, dst_tiles, zerosD):
    cpt = src_tiles.shape[1]
    return pl.kernel(
        _sc_agg_body,
        out_type=jax.ShapeDtypeStruct((NC, NP, D), jnp.float32),
        mesh=_MESH,
        scratch_types=[
            pltpu.VMEM((cpt, CHUNK), jnp.int32),
            pltpu.VMEM((cpt, CHUNK), jnp.int32),
            pltpu.VMEM((CHUNK, D), jnp.float32),
            pltpu.VMEM_SHARED((NP, D), jnp.float32),
            pltpu.SemaphoreType.DMA,
        ],
    )(g, src_tiles, dst_tiles, zerosD)


def _tc_pre_body(degp_ref, x_ref, w_ref, g_ref, dinv_ref):
    d16 = degp_ref[0] + degp_ref[1] + 1.0
    dinv16 = lax.rsqrt(d16)
    dinv_ref[...] = dinv16
    h = jnp.dot(x_ref[...], w_ref[...], preferred_element_type=jnp.float32)
    g_ref[...] = h * dinv16[:, 0:1]


def _tc_pre(degp, xp, W):
    return pl.pallas_call(
        _tc_pre_body,
        out_shape=(
            jax.ShapeDtypeStruct((NP, D), jnp.float32),
            jax.ShapeDtypeStruct((NP, 16), jnp.float32),
        ),
    )(degp, xp, W)


def _postagg(acc_ref, g_ref, dinv_ref, b_ref, gamma_ref, beta_ref):
    dinv = dinv_ref[...][:, 0:1]
    o = (acc_ref[0] + acc_ref[1] + g_ref[...]) * dinv + b_ref[...]
    rid = lax.broadcasted_iota(jnp.int32, (NP, 1), 0)
    valid = (rid < N).astype(jnp.float32)
    mu = jnp.sum(o * valid, axis=0, keepdims=True) * (1.0 / N)
    ex2 = jnp.sum(o * o * valid, axis=0, keepdims=True) * (1.0 / N)
    var = ex2 - mu * mu
    xn = (o - mu) * lax.rsqrt(var + 1e-5) * gamma_ref[...] + beta_ref[...]
    e = jnp.where(xn > 0, xn, jnp.exp(xn) - 1.0)
    return e * valid, dinv


def _tc_mid_body(acc_ref, g_ref, dinv_ref, b_ref, gamma_ref, beta_ref, wn_ref, gn_ref):
    e, dinv = _postagg(acc_ref, g_ref, dinv_ref, b_ref, gamma_ref, beta_ref)
    gn_ref[...] = jnp.dot(e, wn_ref[...], preferred_element_type=jnp.float32) * dinv


def _tc_mid(acc, g, dinv16, b, gamma, beta, Wn):
    return pl.pallas_call(
        _tc_mid_body,
        out_shape=jax.ShapeDtypeStruct((NP, D), jnp.float32),
    )(acc, g, dinv16, b, gamma, beta, Wn)


def _tc_post_body(acc_ref, g_ref, dinv_ref, b_ref, gamma_ref, beta_ref, bi_ref, out_ref):
    e, _ = _postagg(acc_ref, g_ref, dinv_ref, b_ref, gamma_ref, beta_ref)
    bi = bi_ref[...]
    onehot = (bi == lax.broadcasted_iota(jnp.int32, (NP, G), 1)).astype(jnp.float32)
    sums = lax.dot_general(onehot, e, (((0,), (0,)), ((), ())),
                           preferred_element_type=jnp.float32)
    ones_col = jnp.ones((NP, 1), jnp.float32)
    cnt = lax.dot_general(onehot, ones_col, (((0,), (0,)), ((), ())),
                          preferred_element_type=jnp.float32)
    out_ref[...] = sums / jnp.maximum(cnt, 1.0)


def _tc_post(acc, g, dinv16, b, gamma, beta, bi):
    return pl.pallas_call(
        _tc_post_body,
        out_shape=jax.ShapeDtypeStruct((G, D), jnp.float32),
    )(acc, g, dinv16, b, gamma, beta, bi)


def kernel(x, edge_index, batch_idx,
           W0, b0, gamma0, beta0, W1, b1, gamma1, beta1, W2, b2, gamma2, beta2):
    E = edge_index.shape[1]
    cpt = -(-E // (NW * CHUNK))
    ep = NW * cpt * CHUNK
    src = edge_index[0].astype(jnp.int32)
    dst = edge_index[1].astype(jnp.int32)
    pad = jnp.full((ep - E,), N, jnp.int32)
    src_t = jnp.concatenate([src, pad]).reshape(NW, cpt, CHUNK)
    dst_t = jnp.concatenate([dst, pad]).reshape(NW, cpt, CHUNK)
    xp = jnp.zeros((NP, D), jnp.float32).at[:N].set(x)
    bi = jnp.concatenate([batch_idx.astype(jnp.int32),
                          jnp.full((NP - N,), G, jnp.int32)]).reshape(NP, 1)
    zeros16 = jnp.zeros((NP, 16), jnp.float32)
    ones16 = jnp.ones((CHUNK, 16), jnp.float32)
    zerosD = jnp.zeros((NP, D), jnp.float32)

    degp = _sc_deg(dst_t, zeros16, ones16)
    g, dinv16 = _tc_pre(degp, xp, W0)

    acc = _sc_agg(g, src_t, dst_t, zerosD)
    g = _tc_mid(acc, g, dinv16, b0.reshape(1, D), gamma0.reshape(1, D),
                beta0.reshape(1, D), W1)
    acc = _sc_agg(g, src_t, dst_t, zerosD)
    g = _tc_mid(acc, g, dinv16, b1.reshape(1, D), gamma1.reshape(1, D),
                beta1.reshape(1, D), W2)
    acc = _sc_agg(g, src_t, dst_t, zerosD)
    return _tc_post(acc, g, dinv16, b2.reshape(1, D), gamma2.reshape(1, D),
                    beta2.reshape(1, D), bi)
